# baseline (speedup 1.0000x reference)
import jax
import jax.numpy as jnp
from jax import lax
from jax.experimental import pallas as pl
from jax.experimental.pallas import tpu as pltpu
from jax.experimental.pallas import tpu_sc as plsc

_B = 4096
_ND = 13
_NS = 26
_V = 100000
_D = 16
_REP = 64
_H1, _H2 = 256, 128
_NC, _NSUB = 2, 16
_NW = _NC * _NSUB
_BPW = _B // _NW
_LPW = _BPW * _NS
_CH = 416
_NCH = _LPW // _CH
_GROWS = _NS * _V * _D // 128


def _sc_e2_body(gidx_hbm, loff_hbm, e2g_hbm, out_hbm,
                gidx_v, loff_v, grp_v, dst_v, sem_g):
    w = lax.axis_index("s") * _NC + lax.axis_index("c")
    l0 = w * _LPW
    pltpu.sync_copy(gidx_hbm.at[pl.ds(l0, _LPW)], gidx_v)
    pltpu.sync_copy(loff_hbm.at[pl.ds(l0, _LPW)], loff_v)

    iota16 = lax.iota(jnp.int32, 16)

    def per_chunk(ci, carry):
        c0 = ci * _CH
        pltpu.async_copy(e2g_hbm.at[gidx_v.at[pl.ds(c0, _CH)]], grp_v,
                         sem_g).wait()

        def per_16(k0, c2):
            l0 = c0 + k0 * 16
            loff16 = loff_v[pl.ds(l0, 16)]
            rows = k0 * 16 + iota16
            drow = (l0 >> 3) + lax.shift_right_logical(iota16, 3)
            dcol = (iota16 & 7) * _D
            for d in range(_D):
                vals = plsc.load_gather(grp_v, [rows, loff16 + d])
                plsc.store_scatter(dst_v, [drow, dcol + d], vals)
            return c2

        lax.fori_loop(0, _CH // 16, per_16, 0)
        return carry

    lax.fori_loop(0, _NCH, per_chunk, 0)
    pltpu.sync_copy(dst_v, out_hbm.at[pl.ds(w * (_LPW // 8), _LPW // 8)])


def _sc_e2_gather(gidx, loff, e2g):
    return pl.kernel(
        _sc_e2_body,
        out_type=jax.ShapeDtypeStruct((_B * _NS // 8, 128), jnp.float32),
        mesh=plsc.VectorSubcoreMesh(core_axis_name="c", subcore_axis_name="s"),
        scratch_types=[pltpu.VMEM((_LPW,), jnp.int32),
                       pltpu.VMEM((_LPW,), jnp.int32),
                       pltpu.VMEM((_CH, 128), jnp.float32),
                       pltpu.VMEM((_LPW // 8, 128), jnp.float32),
                       pltpu.SemaphoreType.DMA],
        compiler_params=pltpu.CompilerParams(use_tc_tiling_on_sc=True,
                                             needs_layout_passes=False),
    )(gidx, loff, e2g)


def _sc_e1_body(idx_hbm, e1f_hbm, out_hbm, idx_v, scal_v, sem):
    w = lax.axis_index("s") * _NC + lax.axis_index("c")
    l0 = w * _LPW
    pltpu.sync_copy(idx_hbm.at[pl.ds(l0, _LPW)], idx_v)
    pltpu.async_copy(e1f_hbm.at[idx_v], scal_v, sem).wait()
    pltpu.sync_copy(scal_v, out_hbm.at[pl.ds(l0, _LPW)])


def _sc_e1_gather(flat_idx, e1f):
    return pl.kernel(
        _sc_e1_body,
        out_type=jax.ShapeDtypeStruct((_B * _NS,), jnp.float32),
        mesh=plsc.VectorSubcoreMesh(core_axis_name="c", subcore_axis_name="s"),
        scratch_types=[pltpu.VMEM((_LPW,), jnp.int32),
                       pltpu.VMEM((_LPW,), jnp.float32),
                       pltpu.SemaphoreType.DMA],
    )(flat_idx, e1f)


_BLK = 512


def _tc_body(xg_ref, xd_ref, rep_ref, e1g_ref,
             w1a_ref, w1b_ref, w1c_ref, bd1_ref,
             wd2_ref, bd2_ref, wf_ref,
             w1da_ref, w1db_ref, cb_ref,
             out_ref):
    f32 = jnp.float32

    def dot(a, b):
        return lax.dot_general(a, b, (((1,), (0,)), ((), ())),
                               preferred_element_type=f32)

    xg = xg_ref[...]
    xd = xd_ref[...]
    rp = rep_ref[...]
    h1 = dot(xg, w1a_ref[...]) + dot(xd, w1b_ref[...]) + dot(rp, w1c_ref[...])
    h1 = jnp.maximum(h1 + bd1_ref[...], 0.0)
    h2 = jnp.maximum(dot(h1, wd2_ref[...]) + bd2_ref[...], 0.0)
    dnn = dot(h2, wf_ref[...])
    fm1d = dot(xd, w1da_ref[...]) + dot(rp, w1db_ref[...])
    r = lax.broadcasted_iota(jnp.int32, (_NS * _D, _D), 0)
    c = lax.broadcasted_iota(jnp.int32, (_NS * _D, _D), 1)
    m = ((r % _D) == c).astype(f32)
    s = dot(xg, m)
    ssq = dot(xg * xg, m)
    fm2 = 0.5 * jnp.sum(s * s - ssq, axis=1, keepdims=True)
    fm1s = jnp.sum(e1g_ref[...], axis=1, keepdims=True)
    out_ref[...] = dnn + fm1d + fm2 + fm1s + cb_ref[...]


def _tc_dense(xg, xd, rep, e1g, w1a, w1b, w1c, bd1, wd2, bd2, wf, w1da, w1db, cb):
    def blk(shape):
        return pl.BlockSpec(shape, lambda i: (i, 0))

    def full(shape):
        return pl.BlockSpec(shape, lambda i: (0, 0))

    return pl.pallas_call(
        _tc_body,
        grid=(_B // _BLK,),
        in_specs=[blk((_BLK, _NS * _D)), blk((_BLK, _ND)), blk((_BLK, _REP)),
                  blk((_BLK, _NS)),
                  full((_NS * _D, _H1)), full((_ND, _H1)), full((_REP, _H1)),
                  full((1, _H1)),
                  full((_H1, _H2)), full((1, _H2)), full((_H2, 1)),
                  full((_ND, 1)), full((_REP, 1)), full((1, 1))],
        out_specs=blk((_BLK, 1)),
        out_shape=jax.ShapeDtypeStruct((_B, 1), jnp.float32),
    )(xg, xd, rep, e1g, w1a, w1b, w1c, bd1, wd2, bd2, wf, w1da, w1db, cb)


def kernel(representation, target_x, e1, e2, W1d, b1d, Wd1, bd1, Wd2, bd2, Wf, bf):
    sparse_idx = target_x[:, _ND:].astype(jnp.int32)
    flat_idx = (sparse_idx
                + (jnp.arange(_NS, dtype=jnp.int32) * _V)[None, :]).reshape(-1)
    gidx = flat_idx >> 3
    loff = (flat_idx & 7) << 4
    e2g = e2.reshape(_GROWS, 128)
    xg = _sc_e2_gather(gidx, loff, e2g).reshape(_B, _NS * _D)
    e1g = _sc_e1_gather(flat_idx, e1.reshape(_NS * _V)).reshape(_B, _NS)
    xd = target_x[:, :_ND]
    out = _tc_dense(
        xg, xd, representation, e1g,
        Wd1[:_NS * _D], Wd1[_NS * _D:_NS * _D + _ND], Wd1[_NS * _D + _ND:],
        bd1.reshape(1, _H1), Wd2, bd2.reshape(1, _H2), Wf,
        W1d[:_ND], W1d[_ND:], (b1d + bf).reshape(1, 1))
    return out


# speedup vs baseline: 3.0411x; 3.0411x over previous
import jax
import jax.numpy as jnp
from jax import lax
from jax.experimental import pallas as pl
from jax.experimental.pallas import tpu as pltpu
from jax.experimental.pallas import tpu_sc as plsc

_B = 4096
_ND = 13
_NS = 26
_V = 100000
_D = 16
_REP = 64
_H1, _H2 = 256, 128
_NC, _NSUB = 2, 16
_NW = _NC * _NSUB
_BPW = _B // _NW


def _sc_gather_body(idx_hbm, e2t_hbm, e1_hbm, xgt_out, e1gt_out,
                    idx_v, dst_v, e1dst_v, sem):
    w = lax.axis_index("s") * _NC + lax.axis_index("c")
    col0 = w * _BPW
    pltpu.sync_copy(idx_hbm.at[:, pl.ds(col0, _BPW)], idx_v)

    def per_field(f, carry):
        iv = idx_v.at[f]
        cps = [pltpu.async_copy(e2t_hbm.at[f, d].at[iv],
                                dst_v.at[f * _D + d], sem)
               for d in range(_D)]
        cp1 = pltpu.async_copy(e1_hbm.at[f].at[iv], e1dst_v.at[f], sem)
        for cp in cps:
            cp.wait()
        cp1.wait()
        return carry

    lax.fori_loop(0, _NS, per_field, 0)
    pltpu.sync_copy(dst_v, xgt_out.at[:, pl.ds(col0, _BPW)])
    pltpu.sync_copy(e1dst_v, e1gt_out.at[:, pl.ds(col0, _BPW)])


def _sc_gather(idx_t, e2t, e1):
    return pl.kernel(
        _sc_gather_body,
        out_type=(jax.ShapeDtypeStruct((_NS * _D, _B), jnp.float32),
                  jax.ShapeDtypeStruct((_NS, _B), jnp.float32)),
        mesh=plsc.VectorSubcoreMesh(core_axis_name="c", subcore_axis_name="s"),
        scratch_types=[pltpu.VMEM((_NS, _BPW), jnp.int32),
                       pltpu.VMEM((_NS * _D, _BPW), jnp.float32),
                       pltpu.VMEM((_NS, _BPW), jnp.float32),
                       pltpu.SemaphoreType.DMA],
        compiler_params=pltpu.CompilerParams(use_tc_tiling_on_sc=False),
    )(idx_t, e2t, e1)


_CBLK = 512


def _tc_body(xgt_ref, xdt_ref, rpt_ref, e1gt_ref,
             w1at_ref, w1bt_ref, w1ct_ref, bd1c_ref,
             wd2t_ref, bd2c_ref, wft_ref,
             w1dat_ref, w1dbt_ref, cb_ref,
             out_ref):
    f32 = jnp.float32

    def dot(a, b):
        return lax.dot_general(a, b, (((1,), (0,)), ((), ())),
                               preferred_element_type=f32)

    xg = xgt_ref[...]
    xd = xdt_ref[...]
    rp = rpt_ref[...]
    h1 = dot(w1at_ref[...], xg) + dot(w1bt_ref[...], xd) + dot(w1ct_ref[...], rp)
    h1 = jnp.maximum(h1 + bd1c_ref[...], 0.0)
    h2 = jnp.maximum(dot(wd2t_ref[...], h1) + bd2c_ref[...], 0.0)
    dnn = dot(wft_ref[...], h2)
    fm1d = dot(w1dat_ref[...], xd) + dot(w1dbt_ref[...], rp)
    r = lax.broadcasted_iota(jnp.int32, (_D, _NS * _D), 0)
    c = lax.broadcasted_iota(jnp.int32, (_D, _NS * _D), 1)
    m = ((c % _D) == r).astype(f32)
    s = dot(m, xg)
    ssq = dot(m, xg * xg)
    fm2 = 0.5 * jnp.sum(s * s - ssq, axis=0, keepdims=True)
    fm1s = jnp.sum(e1gt_ref[...], axis=0, keepdims=True)
    out_ref[...] = dnn + fm1d + fm2 + fm1s + cb_ref[...]


def _tc_dense(xgt, xdt, rpt, e1gt, w1at, w1bt, w1ct, bd1c, wd2t, bd2c, wft,
              w1dat, w1dbt, cb):
    def blk(nrows):
        return pl.BlockSpec((nrows, _CBLK), lambda i: (0, i))

    def full(shape):
        return pl.BlockSpec(shape, lambda i: (0, 0))

    return pl.pallas_call(
        _tc_body,
        grid=(_B // _CBLK,),
        in_specs=[blk(_NS * _D), blk(_ND), blk(_REP), blk(_NS),
                  full((_H1, _NS * _D)), full((_H1, _ND)), full((_H1, _REP)),
                  full((_H1, 1)),
                  full((_H2, _H1)), full((_H2, 1)), full((1, _H2)),
                  full((1, _ND)), full((1, _REP)), full((1, 1))],
        out_specs=pl.BlockSpec((1, _CBLK), lambda i: (0, i)),
        out_shape=jax.ShapeDtypeStruct((1, _B), jnp.float32),
    )(xgt, xdt, rpt, e1gt, w1at, w1bt, w1ct, bd1c, wd2t, bd2c, wft,
      w1dat, w1dbt, cb)


def kernel(representation, target_x, e1, e2, W1d, b1d, Wd1, bd1, Wd2, bd2, Wf, bf):
    txt = target_x.T
    idx_t = txt[_ND:].astype(jnp.int32)
    e2t = e2.transpose(0, 2, 1)
    xgt, e1gt = _sc_gather(idx_t, e2t, e1)
    outt = _tc_dense(
        xgt, txt[:_ND], representation.T, e1gt,
        Wd1[:_NS * _D].T, Wd1[_NS * _D:_NS * _D + _ND].T, Wd1[_NS * _D + _ND:].T,
        bd1.reshape(_H1, 1), Wd2.T, bd2.reshape(_H2, 1), Wf.T,
        W1d[:_ND].T, W1d[_ND:].T, (b1d + bf).reshape(1, 1))
    return outt.reshape(_B, 1)


# speedup vs baseline: 3.1966x; 1.0511x over previous
import jax
import jax.numpy as jnp
from jax import lax
from jax.experimental import pallas as pl
from jax.experimental.pallas import tpu as pltpu
from jax.experimental.pallas import tpu_sc as plsc

_B = 4096
_ND = 13
_NS = 26
_V = 100000
_D = 16
_REP = 64
_H1, _H2 = 256, 128
_NC, _NSUB = 2, 16
_NW = _NC * _NSUB
_BPW = _B // _NW


def _sc_gather_body(idx_hbm, e2t_hbm, e1_hbm, xgt_out, e1gt_out,
                    idx_v, dst_v, e1dst_v, sem, sem_e1):
    w = lax.axis_index("s") * _NC + lax.axis_index("c")
    col0 = w * _BPW
    pltpu.sync_copy(idx_hbm.at[:, pl.ds(col0, _BPW)], idx_v)

    def per_field(f, carry):
        iv = idx_v.at[f]
        for d in range(_D):
            pltpu.async_copy(e2t_hbm.at[f, d].at[iv], dst_v.at[f * _D + d],
                             sem)
        pltpu.async_copy(e1_hbm.at[f].at[iv], e1dst_v.at[f], sem_e1)
        return carry

    lax.fori_loop(0, _NS, per_field, 0)
    pltpu.make_async_copy(xgt_out.at[:, pl.ds(col0, _BPW)], dst_v, sem).wait()
    pltpu.make_async_copy(e1gt_out.at[:, pl.ds(col0, _BPW)], e1dst_v,
                          sem_e1).wait()
    pltpu.sync_copy(dst_v, xgt_out.at[:, pl.ds(col0, _BPW)])
    pltpu.sync_copy(e1dst_v, e1gt_out.at[:, pl.ds(col0, _BPW)])


def _sc_gather(idx_t, e2t, e1):
    return pl.kernel(
        _sc_gather_body,
        out_type=(jax.ShapeDtypeStruct((_NS * _D, _B), jnp.float32),
                  jax.ShapeDtypeStruct((_NS, _B), jnp.float32)),
        mesh=plsc.VectorSubcoreMesh(core_axis_name="c", subcore_axis_name="s"),
        scratch_types=[pltpu.VMEM((_NS, _BPW), jnp.int32),
                       pltpu.VMEM((_NS * _D, _BPW), jnp.float32),
                       pltpu.VMEM((_NS, _BPW), jnp.float32),
                       pltpu.SemaphoreType.DMA,
                       pltpu.SemaphoreType.DMA],
        compiler_params=pltpu.CompilerParams(use_tc_tiling_on_sc=False),
    )(idx_t, e2t, e1)


_CBLK = 512


def _tc_body(xgt_ref, xdt_ref, rpt_ref, e1gt_ref,
             w1at_ref, w1bt_ref, w1ct_ref, bd1c_ref,
             wd2t_ref, bd2c_ref, wft_ref,
             w1dat_ref, w1dbt_ref, cb_ref,
             out_ref):
    f32 = jnp.float32

    def dot(a, b):
        return lax.dot_general(a, b, (((1,), (0,)), ((), ())),
                               preferred_element_type=f32)

    xg = xgt_ref[...]
    xd = xdt_ref[...]
    rp = rpt_ref[...]
    h1 = dot(w1at_ref[...], xg) + dot(w1bt_ref[...], xd) + dot(w1ct_ref[...], rp)
    h1 = jnp.maximum(h1 + bd1c_ref[...], 0.0)
    h2 = jnp.maximum(dot(wd2t_ref[...], h1) + bd2c_ref[...], 0.0)
    dnn = dot(wft_ref[...], h2)
    fm1d = dot(w1dat_ref[...], xd) + dot(w1dbt_ref[...], rp)
    r = lax.broadcasted_iota(jnp.int32, (_D, _NS * _D), 0)
    c = lax.broadcasted_iota(jnp.int32, (_D, _NS * _D), 1)
    m = ((c % _D) == r).astype(f32)
    s = dot(m, xg)
    ssq = dot(m, xg * xg)
    fm2 = 0.5 * jnp.sum(s * s - ssq, axis=0, keepdims=True)
    fm1s = jnp.sum(e1gt_ref[...], axis=0, keepdims=True)
    out_ref[...] = dnn + fm1d + fm2 + fm1s + cb_ref[...]


def _tc_dense(xgt, xdt, rpt, e1gt, w1at, w1bt, w1ct, bd1c, wd2t, bd2c, wft,
              w1dat, w1dbt, cb):
    def blk(nrows):
        return pl.BlockSpec((nrows, _CBLK), lambda i: (0, i))

    def full(shape):
        return pl.BlockSpec(shape, lambda i: (0, 0))

    return pl.pallas_call(
        _tc_body,
        grid=(_B // _CBLK,),
        in_specs=[blk(_NS * _D), blk(_ND), blk(_REP), blk(_NS),
                  full((_H1, _NS * _D)), full((_H1, _ND)), full((_H1, _REP)),
                  full((_H1, 1)),
                  full((_H2, _H1)), full((_H2, 1)), full((1, _H2)),
                  full((1, _ND)), full((1, _REP)), full((1, 1))],
        out_specs=pl.BlockSpec((1, _CBLK), lambda i: (0, i)),
        out_shape=jax.ShapeDtypeStruct((1, _B), jnp.float32),
    )(xgt, xdt, rpt, e1gt, w1at, w1bt, w1ct, bd1c, wd2t, bd2c, wft,
      w1dat, w1dbt, cb)


def kernel(representation, target_x, e1, e2, W1d, b1d, Wd1, bd1, Wd2, bd2, Wf, bf):
    txt = target_x.T
    idx_t = txt[_ND:].astype(jnp.int32)
    e2t = e2.transpose(0, 2, 1)
    xgt, e1gt = _sc_gather(idx_t, e2t, e1)
    outt = _tc_dense(
        xgt, txt[:_ND], representation.T, e1gt,
        Wd1[:_NS * _D].T, Wd1[_NS * _D:_NS * _D + _ND].T, Wd1[_NS * _D + _ND:].T,
        bd1.reshape(_H1, 1), Wd2.T, bd2.reshape(_H2, 1), Wf.T,
        W1d[:_ND].T, W1d[_ND:].T, (b1d + bf).reshape(1, 1))
    return outt.reshape(_B, 1)
